# Initial kernel scaffold; baseline (speedup 1.0000x reference)
import jax
import jax.numpy as jnp
from jax.experimental import pallas as pl


def kernel(x, edge_index, batch, proj_W, proj_b, bn1_g, bn1_b, ggc_W, gru_Wih, gru_Whh, gru_bih, gru_bhh, bn2_g, bn2_b, fc1_W, fc1_b, bn3_g, bn3_b, fc2_W, fc2_b):
    raise NotImplementedError("write your pallas kernel here")


# speedup vs baseline: 3.1600x; 3.1600x over previous
import functools

import jax
import jax.numpy as jnp
from jax import lax
from jax.experimental import pallas as pl
from jax.experimental.pallas import tpu as pltpu
from jax.experimental.pallas import tpu_sc as plsc

_N = 10000
_E = 320000
_H = 128
_G = 32
_STEPS = 3
_EPS = 1e-5

_NP = 10240
_NW = 32
_EPW = _E // _NW
_CH = 80
_NCH = _EPW // _CH
_RPT = _NP // 16


def _mm(a, b, ca, cb):
    return jax.lax.dot_general(
        a, b, (((ca,), (cb,)), ((), ())),
        preferred_element_type=jnp.float32)


def _bn_rows(h, g, b):
    mu = jnp.mean(h, axis=0, keepdims=True)
    var = jnp.mean((h - mu) ** 2, axis=0, keepdims=True)
    return (h - mu) / jnp.sqrt(var + _EPS) * g + b


def _proj_body(x_ref, w_ref, b_ref, g_ref, bb_ref, o_ref):
    h = _mm(x_ref[...], w_ref[...], 1, 1) + b_ref[...]
    h = _bn_rows(h, g_ref[...], bb_ref[...])
    o_ref[...] = jnp.maximum(h, 0.0)


def _pre_body(h_ref, wi_ref, whh_ref, bhh_ref, m_ref, gh_ref):
    h = h_ref[...]
    m_ref[...] = _mm(h, wi_ref[...], 1, 0)
    gh_ref[...] = _mm(h, whh_ref[...], 1, 1) + bhh_ref[...]


def _gru_body(aggp_ref, gh_ref, h_ref, wih_ref, bih_ref, o_ref):
    a = aggp_ref[0] + aggp_ref[1]
    gi = _mm(a, wih_ref[...], 1, 1) + bih_ref[...]
    gh = gh_ref[...]
    h = h_ref[...]
    r = jax.nn.sigmoid(gi[:, :_H] + gh[:, :_H])
    z = jax.nn.sigmoid(gi[:, _H:2 * _H] + gh[:, _H:2 * _H])
    n = jnp.tanh(gi[:, 2 * _H:] + r * gh[:, 2 * _H:])
    o_ref[...] = (1.0 - z) * n + z * h


def _bn2_body(h_ref, g2_ref, b2_ref, o_ref):
    h = _bn_rows(h_ref[...], g2_ref[...], b2_ref[...])
    o_ref[...] = jnp.maximum(h, 0.0)


def _pool_body(h_ref, b_ref, sum_ref, mx_ref, cnt_ref):
    gidx = pl.program_id(0)
    mask = b_ref[...] == gidx
    h = h_ref[...]
    s = jnp.sum(jnp.where(mask, h, 0.0), axis=0, keepdims=True)
    sum_ref[...] = s.reshape(1, 1, _H)
    m = jnp.max(jnp.where(mask, h, -3.0e38), axis=0, keepdims=True)
    mx_ref[...] = m.reshape(1, 1, _H)
    cnt = jnp.sum(jnp.where(mask, 1.0, 0.0), axis=0, keepdims=True)
    cnt_ref[...] = jnp.broadcast_to(jnp.sum(cnt, axis=1, keepdims=True),
                                    (1, _H)).reshape(1, 1, _H)


def _mm_def(a, b, ca, cb):
    return jax.lax.dot_general(
        a, b, (((ca,), (cb,)), ((), ())),
        preferred_element_type=jnp.float32)


def _head_body(sum_ref, mx_ref, cnt_ref, w1_ref, b1_ref,
               g3_ref, b3_ref, w2_ref, bf_ref, o_ref):
    counts = cnt_ref[...]
    mean = sum_ref[...] / jnp.maximum(counts, 1.0)
    mx = jnp.where(counts > 0.0, mx_ref[...], 0.0)
    gcat = jnp.concatenate([mean, mx], axis=1)
    y = _mm_def(gcat, w1_ref[...], 1, 1) + b1_ref[...]
    mu = jnp.mean(y, axis=0, keepdims=True)
    var = jnp.mean((y - mu) ** 2, axis=0, keepdims=True)
    y = (y - mu) / jnp.sqrt(var + _EPS) * g3_ref[...] + b3_ref[...]
    y = jnp.maximum(y, 0.0)
    o_ref[...] = _mm_def(y, w2_ref[...], 1, 1) + bf_ref[...]


def _tc_call(body, out_shapes, *args):
    return pl.pallas_call(
        body,
        out_shape=out_shapes,
    )(*args)


_BLK = 2000


def _row_block(shape):
    blk = shape[:-2] + (_BLK, shape[-1])
    nd = len(shape)

    def imap(i):
        return (0,) * (nd - 2) + (i, 0)

    return pl.BlockSpec(blk, imap)


def _full_block(shape):
    nd = len(shape)
    return pl.BlockSpec(shape, lambda i: (0,) * nd)


def _tc_rowcall(body, out_shapes, row_args, full_args):
    in_specs = ([_row_block(a.shape) for a in row_args]
                + [_full_block(a.shape) for a in full_args])
    outs = (out_shapes if isinstance(out_shapes, (tuple, list))
            else [out_shapes])
    out_specs = [_row_block(o.shape) for o in outs]
    res = pl.pallas_call(
        body,
        grid=(_N // _BLK,),
        in_specs=in_specs,
        out_specs=out_specs if isinstance(out_shapes, (tuple, list))
        else out_specs[0],
        out_shape=out_shapes,
    )(*row_args, *full_args)
    return res


@functools.cache
def _make_sc_scatter():
    mesh = plsc.VectorSubcoreMesh(core_axis_name="c", subcore_axis_name="s")
    return pl.kernel(
        _sc_scatter_body,
        mesh=mesh,
        out_type=jax.ShapeDtypeStruct((2, _NP, _H), jnp.float32),
        scratch_types=[
            pltpu.VMEM((_CH,), jnp.int32),
            pltpu.VMEM((_CH,), jnp.int32),
            pltpu.VMEM((_CH, _H), jnp.float32),
            pltpu.VMEM_SHARED((_NP, _H), jnp.float32),
            pltpu.SemaphoreType.DMA,
        ],
    )


def _sc_scatter_body(m_hbm, src_hbm, dst_hbm, out_hbm, src_v, dst_v, rows_v,
                     acc, sem):
    cid = lax.axis_index("c")
    sid = lax.axis_index("s")
    wid = sid * 2 + cid

    zeros16 = jnp.zeros((16,), jnp.float32)

    def zbody(i, carry):
        for j in range(_H // 16):
            rows_v[i, pl.ds(j * 16, 16)] = zeros16
        return carry

    lax.fori_loop(0, _CH, zbody, 0)
    for j in range(_RPT // _CH):
        pltpu.sync_copy(rows_v, acc.at[pl.ds(sid * _RPT + j * _CH, _CH)])
    plsc.subcore_barrier()

    def body(k, carry):
        base = wid * _EPW + k * _CH
        pltpu.sync_copy(src_hbm.at[pl.ds(base, _CH)], src_v)
        pltpu.sync_copy(dst_hbm.at[pl.ds(base, _CH)], dst_v)
        pltpu.async_copy(m_hbm.at[src_v], rows_v, sem).wait()
        pltpu.sync_copy(rows_v, acc.at[dst_v], add=True)
        return carry

    lax.fori_loop(0, _NCH, body, 0)
    plsc.subcore_barrier()
    pltpu.sync_copy(acc.at[pl.ds(sid * _RPT, _RPT)],
                    out_hbm.at[cid].at[pl.ds(sid * _RPT, _RPT)])


def kernel(x, edge_index, batch, proj_W, proj_b, bn1_g, bn1_b, ggc_W,
           gru_Wih, gru_Whh, gru_bih, gru_bhh, bn2_g, bn2_b,
           fc1_W, fc1_b, bn3_g, bn3_b, fc2_W, fc2_b):
    src = edge_index[0].astype(jnp.int32)
    dst = edge_index[1].astype(jnp.int32)
    order = jnp.argsort(dst, stable=True)
    src = src[order]
    dst = dst[order]
    batch2 = batch.astype(jnp.int32).reshape(_N, 1)
    f32 = jnp.float32

    h = _tc_call(
        _proj_body, jax.ShapeDtypeStruct((_N, _H), f32),
        x, proj_W, proj_b.reshape(1, _H), bn1_g.reshape(1, _H),
        bn1_b.reshape(1, _H))

    for i in range(_STEPS):
        m, gh = _tc_rowcall(
            _pre_body,
            (jax.ShapeDtypeStruct((_N, _H), f32),
             jax.ShapeDtypeStruct((_N, 3 * _H), f32)),
            [h], [ggc_W[i], gru_Whh, gru_bhh.reshape(1, 3 * _H)])
        aggp = _make_sc_scatter()(m, src, dst)
        h = _tc_rowcall(
            _gru_body, jax.ShapeDtypeStruct((_N, _H), f32),
            [aggp, gh, h], [gru_Wih, gru_bih.reshape(1, 3 * _H)])

    h = _tc_call(
        _bn2_body, jax.ShapeDtypeStruct((_N, _H), f32),
        h, bn2_g.reshape(1, _H), bn2_b.reshape(1, _H))

    pool_out = (jax.ShapeDtypeStruct((_G, 1, _H), f32),
                jax.ShapeDtypeStruct((_G, 1, _H), f32),
                jax.ShapeDtypeStruct((_G, 1, _H), f32))
    sums, mx, cnt = pl.pallas_call(
        _pool_body,
        grid=(_G,),
        in_specs=[pl.BlockSpec((_N, _H), lambda g: (0, 0)),
                  pl.BlockSpec((_N, 1), lambda g: (0, 0))],
        out_specs=[pl.BlockSpec((1, 1, _H), lambda g: (g, 0, 0))] * 3,
        out_shape=pool_out,
    )(h, batch2)
    sums = sums.reshape(_G, _H)
    mx = mx.reshape(_G, _H)
    cnt = cnt.reshape(_G, _H)

    out = _tc_call(
        _head_body, jax.ShapeDtypeStruct((_G, 2), f32),
        sums, mx, cnt, fc1_W, fc1_b.reshape(1, _H), bn3_g.reshape(1, _H),
        bn3_b.reshape(1, _H), fc2_W, fc2_b.reshape(1, 2))
    return out


# speedup vs baseline: 4.6697x; 1.4778x over previous
import functools

import jax
import jax.numpy as jnp
from jax import lax
from jax.experimental import pallas as pl
from jax.experimental.pallas import tpu as pltpu
from jax.experimental.pallas import tpu_sc as plsc

_N = 10000
_E = 320000
_H = 128
_G = 32
_STEPS = 3
_EPS = 1e-5

_NP = 10240
_NW = 32
_EPW = _E // _NW
_CH = 80
_NCH = _EPW // _CH
_RPT = _NP // 16


def _mm(a, b, ca, cb):
    return jax.lax.dot_general(
        a, b, (((ca,), (cb,)), ((), ())),
        preferred_element_type=jnp.float32)


def _bn_rows(h, g, b):
    mu = jnp.mean(h, axis=0, keepdims=True)
    var = jnp.mean((h - mu) ** 2, axis=0, keepdims=True)
    return (h - mu) / jnp.sqrt(var + _EPS) * g + b


def _proj_body(x_ref, w_ref, b_ref, g_ref, bb_ref, o_ref):
    h = _mm(x_ref[...], w_ref[...], 1, 1) + b_ref[...]
    h = _bn_rows(h, g_ref[...], bb_ref[...])
    o_ref[...] = jnp.maximum(h, 0.0)


def _pre_body(h_ref, wi_ref, whh_ref, bhh_ref, m_ref, gh_ref):
    h = h_ref[...]
    m_ref[...] = _mm(h, wi_ref[...], 1, 0)
    gh_ref[...] = _mm(h, whh_ref[...], 1, 1) + bhh_ref[...]


def _gru_body(aggp_ref, gh_ref, h_ref, wih_ref, bih_ref, o_ref):
    a = aggp_ref[0] + aggp_ref[1]
    gi = _mm(a, wih_ref[...], 1, 1) + bih_ref[...]
    gh = gh_ref[...]
    h = h_ref[...]
    r = jax.nn.sigmoid(gi[:, :_H] + gh[:, :_H])
    z = jax.nn.sigmoid(gi[:, _H:2 * _H] + gh[:, _H:2 * _H])
    n = jnp.tanh(gi[:, 2 * _H:] + r * gh[:, 2 * _H:])
    o_ref[...] = (1.0 - z) * n + z * h


def _bn2_body(h_ref, g2_ref, b2_ref, o_ref):
    h = _bn_rows(h_ref[...], g2_ref[...], b2_ref[...])
    o_ref[...] = jnp.maximum(h, 0.0)


def _pool_body(h_ref, b_ref, sum_ref, mx_ref, cnt_ref):
    gidx = pl.program_id(0)
    mask = b_ref[...] == gidx
    h = h_ref[...]
    s = jnp.sum(jnp.where(mask, h, 0.0), axis=0, keepdims=True)
    sum_ref[...] = s.reshape(1, 1, _H)
    m = jnp.max(jnp.where(mask, h, -3.0e38), axis=0, keepdims=True)
    mx_ref[...] = m.reshape(1, 1, _H)
    cnt = jnp.sum(jnp.where(mask, 1.0, 0.0), axis=0, keepdims=True)
    cnt_ref[...] = jnp.broadcast_to(jnp.sum(cnt, axis=1, keepdims=True),
                                    (1, _H)).reshape(1, 1, _H)


def _mm_def(a, b, ca, cb):
    return jax.lax.dot_general(
        a, b, (((ca,), (cb,)), ((), ())),
        preferred_element_type=jnp.float32)


def _head_body(sum_ref, mx_ref, cnt_ref, w1_ref, b1_ref,
               g3_ref, b3_ref, w2_ref, bf_ref, o_ref):
    counts = cnt_ref[...]
    mean = sum_ref[...] / jnp.maximum(counts, 1.0)
    mx = jnp.where(counts > 0.0, mx_ref[...], 0.0)
    gcat = jnp.concatenate([mean, mx], axis=1)
    y = _mm_def(gcat, w1_ref[...], 1, 1) + b1_ref[...]
    mu = jnp.mean(y, axis=0, keepdims=True)
    var = jnp.mean((y - mu) ** 2, axis=0, keepdims=True)
    y = (y - mu) / jnp.sqrt(var + _EPS) * g3_ref[...] + b3_ref[...]
    y = jnp.maximum(y, 0.0)
    o_ref[...] = _mm_def(y, w2_ref[...], 1, 1) + bf_ref[...]


def _tc_call(body, out_shapes, *args):
    return pl.pallas_call(
        body,
        out_shape=out_shapes,
    )(*args)


_BLK = 2000


def _row_block(shape):
    blk = shape[:-2] + (_BLK, shape[-1])
    nd = len(shape)

    def imap(i):
        return (0,) * (nd - 2) + (i, 0)

    return pl.BlockSpec(blk, imap)


def _full_block(shape):
    nd = len(shape)
    return pl.BlockSpec(shape, lambda i: (0,) * nd)


def _tc_rowcall(body, out_shapes, row_args, full_args):
    in_specs = ([_row_block(a.shape) for a in row_args]
                + [_full_block(a.shape) for a in full_args])
    outs = (out_shapes if isinstance(out_shapes, (tuple, list))
            else [out_shapes])
    out_specs = [_row_block(o.shape) for o in outs]
    res = pl.pallas_call(
        body,
        grid=(_N // _BLK,),
        in_specs=in_specs,
        out_specs=out_specs if isinstance(out_shapes, (tuple, list))
        else out_specs[0],
        out_shape=out_shapes,
    )(*row_args, *full_args)
    return res


@functools.cache
def _make_sc_scatter():
    mesh = plsc.VectorSubcoreMesh(core_axis_name="c", subcore_axis_name="s")
    return pl.kernel(
        _sc_scatter_body,
        mesh=mesh,
        out_type=jax.ShapeDtypeStruct((2, _NP, _H), jnp.float32),
        scratch_types=[
            pltpu.VMEM((_EPW,), jnp.int32),
            pltpu.VMEM((_CH,), jnp.int32),
            pltpu.VMEM((_CH,), jnp.int32),
            pltpu.VMEM((2, _CH, _H), jnp.float32),
            pltpu.VMEM_SHARED((_NP, _H), jnp.float32),
            pltpu.SemaphoreType.DMA,
            pltpu.SemaphoreType.DMA,
        ],
    )


def _sc_scatter_body(m_hbm, src_hbm, dst_hbm, out_hbm, src_v, dst_v0, dst_v1,
                     rows_v, acc, sem_a, sem_b):
    cid = lax.axis_index("c")
    sid = lax.axis_index("s")
    wid = sid * 2 + cid

    zeros16 = jnp.zeros((16,), jnp.float32)

    def zbody(i, carry):
        for j in range(_H // 16):
            rows_v[0, i, pl.ds(j * 16, 16)] = zeros16
        return carry

    lax.fori_loop(0, _CH, zbody, 0)
    for j in range(_RPT // _CH):
        pltpu.sync_copy(rows_v.at[0], acc.at[pl.ds(sid * _RPT + j * _CH, _CH)])
    plsc.subcore_barrier()

    pltpu.sync_copy(src_hbm.at[pl.ds(wid * _EPW, _EPW)], src_v)

    def gather(c, buf, sem):
        return pltpu.async_copy(
            m_hbm.at[src_v.at[pl.ds(c * _CH, _CH)]], rows_v.at[buf], sem)

    def gwait(c, buf, sem):
        pltpu.make_async_copy(
            m_hbm.at[src_v.at[pl.ds(c * _CH, _CH)]], rows_v.at[buf],
            sem).wait()

    def scat(c, buf, dv):
        pltpu.sync_copy(dst_hbm.at[pl.ds(wid * _EPW + c * _CH, _CH)], dv)
        pltpu.sync_copy(rows_v.at[buf], acc.at[dv], add=True)

    gather(0, 0, sem_a)
    gather(1, 1, sem_b)

    def body(j, carry):
        c0 = 2 * j
        gwait(c0, 0, sem_a)
        scat(c0, 0, dst_v0)
        gather(c0 + 2, 0, sem_a)
        gwait(c0 + 1, 1, sem_b)
        scat(c0 + 1, 1, dst_v1)
        gather(jnp.minimum(c0 + 3, _NCH - 1), 1, sem_b)
        return carry

    lax.fori_loop(0, (_NCH - 1) // 2, body, 0)
    gwait(_NCH - 1, 0, sem_a)
    scat(_NCH - 1, 0, dst_v0)
    gwait(_NCH - 1, 1, sem_b)
    plsc.subcore_barrier()
    pltpu.sync_copy(acc.at[pl.ds(sid * _RPT, _RPT)],
                    out_hbm.at[cid].at[pl.ds(sid * _RPT, _RPT)])


def kernel(x, edge_index, batch, proj_W, proj_b, bn1_g, bn1_b, ggc_W,
           gru_Wih, gru_Whh, gru_bih, gru_bhh, bn2_g, bn2_b,
           fc1_W, fc1_b, bn3_g, bn3_b, fc2_W, fc2_b):
    src = edge_index[0].astype(jnp.int32)
    dst = edge_index[1].astype(jnp.int32)
    order = jnp.argsort(dst, stable=True)
    src = src[order]
    dst = dst[order]
    batch2 = batch.astype(jnp.int32).reshape(_N, 1)
    f32 = jnp.float32

    h = _tc_call(
        _proj_body, jax.ShapeDtypeStruct((_N, _H), f32),
        x, proj_W, proj_b.reshape(1, _H), bn1_g.reshape(1, _H),
        bn1_b.reshape(1, _H))

    for i in range(_STEPS):
        m, gh = _tc_rowcall(
            _pre_body,
            (jax.ShapeDtypeStruct((_N, _H), f32),
             jax.ShapeDtypeStruct((_N, 3 * _H), f32)),
            [h], [ggc_W[i], gru_Whh, gru_bhh.reshape(1, 3 * _H)])
        aggp = _make_sc_scatter()(m, src, dst)
        h = _tc_rowcall(
            _gru_body, jax.ShapeDtypeStruct((_N, _H), f32),
            [aggp, gh, h], [gru_Wih, gru_bih.reshape(1, 3 * _H)])

    h = _tc_call(
        _bn2_body, jax.ShapeDtypeStruct((_N, _H), f32),
        h, bn2_g.reshape(1, _H), bn2_b.reshape(1, _H))

    pool_out = (jax.ShapeDtypeStruct((_G, 1, _H), f32),
                jax.ShapeDtypeStruct((_G, 1, _H), f32),
                jax.ShapeDtypeStruct((_G, 1, _H), f32))
    sums, mx, cnt = pl.pallas_call(
        _pool_body,
        grid=(_G,),
        in_specs=[pl.BlockSpec((_N, _H), lambda g: (0, 0)),
                  pl.BlockSpec((_N, 1), lambda g: (0, 0))],
        out_specs=[pl.BlockSpec((1, 1, _H), lambda g: (g, 0, 0))] * 3,
        out_shape=pool_out,
    )(h, batch2)
    sums = sums.reshape(_G, _H)
    mx = mx.reshape(_G, _H)
    cnt = cnt.reshape(_G, _H)

    out = _tc_call(
        _head_body, jax.ShapeDtypeStruct((_G, 2), f32),
        sums, mx, cnt, fc1_W, fc1_b.reshape(1, _H), bn3_g.reshape(1, _H),
        bn3_b.reshape(1, _H), fc2_W, fc2_b.reshape(1, 2))
    return out
